# Initial kernel scaffold; baseline (speedup 1.0000x reference)
import jax
import jax.numpy as jnp
from jax.experimental import pallas as pl


def kernel(x, edge_index, U, psi, gat_W, gat_att_src, gat_att_dst, gat_bias, mlp_W, mlp_b, out_W, out_b):
    raise NotImplementedError("write your pallas kernel here")


# speedup vs baseline: 4.2367x; 4.2367x over previous
import functools

import jax
import jax.numpy as jnp
from jax import lax
from jax.experimental import pallas as pl
from jax.experimental.pallas import tpu as pltpu

N = 2048
E = 32768
F_IN = 128
HEADS = 2
NHID = 64
NCLS = 16
NGAT = 13
NPSI = 3
E_TOT = E + N

RB = 256


def _mmu_body(u_ref, r_ref, o_ref):
    o_ref[...] = jnp.dot(u_ref[...], jnp.abs(r_ref[...]),
                         preferred_element_type=jnp.float32)


def _mm_u_abs(U, rhs):
    F = rhs.shape[1]
    nc = F // 128
    grid = (N // RB, nc)
    return pl.pallas_call(
        _mmu_body,
        grid=grid,
        in_specs=[
            pl.BlockSpec((RB, N), lambda r, c: (r, 0)),
            pl.BlockSpec((N, 128), lambda r, c: (0, c)),
        ],
        out_specs=pl.BlockSpec((RB, 128), lambda r, c: (r, c)),
        out_shape=jax.ShapeDtypeStruct((N, F), jnp.float32),
    )(U, rhs)


def _mmp_body(p_ref, r_ref, o_ref):
    o_ref[...] = jnp.dot(p_ref[0], jnp.abs(r_ref[...]),
                         preferred_element_type=jnp.float32)


def _mm_psi_abs(psi, rhs):
    F = rhs.shape[1]
    nc = F // 128
    grid = (NPSI, N // RB, nc)
    return pl.pallas_call(
        _mmp_body,
        grid=grid,
        in_specs=[
            pl.BlockSpec((1, RB, N), lambda j, r, c: (j, r, 0)),
            pl.BlockSpec((N, 128), lambda j, r, c: (0, c)),
        ],
        out_specs=pl.BlockSpec((RB, 128), lambda j, r, c: (r, j * nc + c)),
        out_shape=jax.ShapeDtypeStruct((N, NPSI * F), jnp.float32),
    )(psi, rhs)


def _prep_body(coef_ref, w_ref, asrc_ref, adst_ref, h_ref, a_ref, b_ref, c_ref):
    r = pl.program_id(1)
    h = jnp.dot(coef_ref[...], w_ref[0], preferred_element_type=jnp.float32)
    h_ref[0] = h
    a_s = asrc_ref[0]
    a_d = adst_ref[0]
    h2 = h.reshape(RB, HEADS, F_IN)
    al_s = jnp.sum(h2 * a_s[None], axis=-1)
    al_d = jnp.sum(h2 * a_d[None], axis=-1)
    a_ref[0] = al_s
    b_ref[0] = al_d

    ms = jnp.broadcast_to(jnp.max(al_s, axis=0)[:, None], (HEADS, 16))
    md = jnp.broadcast_to(jnp.max(al_d, axis=0)[:, None], (HEADS, 16))

    @pl.when(r == 0)
    def _():
        c_ref[0, :, 0, :] = ms
        c_ref[0, :, 1, :] = md

    @pl.when(r != 0)
    def _():
        c_ref[0, :, 0, :] = jnp.maximum(c_ref[0, :, 0, :], ms)
        c_ref[0, :, 1, :] = jnp.maximum(c_ref[0, :, 1, :], md)


def _gat_prep(coef_all, gat_W, att_src, att_dst):
    grid = (NGAT, N // RB)
    return pl.pallas_call(
        _prep_body,
        grid=grid,
        in_specs=[
            pl.BlockSpec((RB, F_IN), lambda l, r: (r, l)),
            pl.BlockSpec((1, F_IN, HEADS * F_IN), lambda l, r: (l, 0, 0)),
            pl.BlockSpec((1, HEADS, F_IN), lambda l, r: (l, 0, 0)),
            pl.BlockSpec((1, HEADS, F_IN), lambda l, r: (l, 0, 0)),
        ],
        out_specs=[
            pl.BlockSpec((1, RB, HEADS * F_IN), lambda l, r: (l, r, 0)),
            pl.BlockSpec((1, RB, HEADS), lambda l, r: (l, r, 0)),
            pl.BlockSpec((1, RB, HEADS), lambda l, r: (l, r, 0)),
            pl.BlockSpec((1, HEADS, 2, 16), lambda l, r: (l, 0, 0, 0)),
        ],
        out_shape=[
            jax.ShapeDtypeStruct((NGAT, N, HEADS * F_IN), jnp.float32),
            jax.ShapeDtypeStruct((NGAT, N, HEADS), jnp.float32),
            jax.ShapeDtypeStruct((NGAT, N, HEADS), jnp.float32),
            jax.ShapeDtypeStruct((NGAT, HEADS, 2, 16), jnp.float32),
        ],
    )(coef_all, gat_W, att_src, att_dst)


def _elu(x):
    return jnp.where(x > 0, x, jnp.exp(jnp.minimum(x, 0.0)) - 1.0)


def _head_body(acc_ref, s_ref, bias_ref, mw_ref, mb_ref, ow_ref, ob_ref, o_ref):
    ys = []
    for l in range(NGAT):
        a = acc_ref[l]
        s = s_ref[l]
        d0 = jnp.broadcast_to(s[:, 0:1], (RB, F_IN)) + 1e-16
        d1 = jnp.broadcast_to(s[:, 1:2], (RB, F_IN)) + 1e-16
        g = a / jnp.concatenate([d0, d1], axis=1) + bias_ref[l][None, :]
        y = jnp.dot(_elu(g), mw_ref[l], preferred_element_type=jnp.float32)
        ys.append(y + mb_ref[l][None, :])
    yo = jnp.concatenate(ys, axis=1)
    logits = jnp.dot(_elu(yo), ow_ref[...],
                     preferred_element_type=jnp.float32) + ob_ref[...][None, :]
    m = jnp.max(logits, axis=1, keepdims=True)
    z = logits - m
    o_ref[...] = z - jnp.log(jnp.sum(jnp.exp(z), axis=1, keepdims=True))


def _head(acc_all, s_all, gat_bias, mlp_W, mlp_b, out_W, out_b):
    grid = (N // RB,)
    return pl.pallas_call(
        _head_body,
        grid=grid,
        in_specs=[
            pl.BlockSpec((NGAT, RB, HEADS * F_IN), lambda r: (0, r, 0)),
            pl.BlockSpec((NGAT, RB, 16), lambda r: (0, r, 0)),
            pl.BlockSpec((NGAT, HEADS * F_IN), lambda r: (0, 0)),
            pl.BlockSpec((NGAT, HEADS * F_IN, NHID), lambda r: (0, 0, 0)),
            pl.BlockSpec((NGAT, NHID), lambda r: (0, 0)),
            pl.BlockSpec((NGAT * NHID, NCLS), lambda r: (0, 0)),
            pl.BlockSpec((NCLS,), lambda r: (0,)),
        ],
        out_specs=pl.BlockSpec((RB, NCLS), lambda r: (r, 0)),
        out_shape=jax.ShapeDtypeStruct((N, NCLS), jnp.float32),
    )(acc_all, s_all, gat_bias, mlp_W, mlp_b, out_W, out_b)


def _edge_phase(h_all, asrc_all, adst_all, cmax_all, src, dst):
    C = jnp.maximum(0.0, cmax_all[:, :, 0, 0] + cmax_all[:, :, 1, 0])
    z = asrc_all[:, src, :] + adst_all[:, dst, :]
    e = jnp.where(z > 0, z, 0.2 * z)
    w = jnp.exp(e - C[:, None, :])
    s = jax.vmap(lambda wl: jax.ops.segment_sum(wl, dst, num_segments=N))(w)
    s16 = jnp.concatenate(
        [s, jnp.zeros((NGAT, N, 16 - HEADS), jnp.float32)], axis=-1)
    hh = h_all.reshape(NGAT, N, HEADS, F_IN)
    msg = hh[:, src] * w[..., None]
    acc = jax.vmap(lambda ml: jax.ops.segment_sum(ml, dst, num_segments=N))(msg)
    return acc.reshape(NGAT, N, HEADS * F_IN), s16


_PERM9 = [0, 3, 6, 1, 4, 7, 2, 5, 8]


def kernel(x, edge_index, U, psi, gat_W, gat_att_src, gat_att_dst, gat_bias,
           mlp_W, mlp_b, out_W, out_b):
    loop = jnp.arange(N, dtype=edge_index.dtype)
    ei = jnp.concatenate([edge_index, jnp.stack([loop, loop])], axis=1)
    src, dst = ei[0], ei[1]

    low = _mm_u_abs(U, x)
    T = _mm_psi_abs(psi, x)
    coef1 = _mm_u_abs(U, T)
    T2 = _mm_psi_abs(psi, T)
    coef2 = _mm_u_abs(U, T2)
    coef2r = coef2.reshape(N, 9, F_IN)[:, jnp.array(_PERM9), :].reshape(N, 9 * F_IN)
    coef_all = jnp.concatenate([low, coef1, coef2r], axis=1)

    h_all, asrc_all, adst_all, cmax_all = _gat_prep(
        coef_all, gat_W, gat_att_src, gat_att_dst)

    acc_all, s_all = _edge_phase(h_all, asrc_all, adst_all, cmax_all, src, dst)

    return _head(acc_all, s_all, gat_bias, mlp_W, mlp_b, out_W, out_b)


# speedup vs baseline: 11.0076x; 2.5981x over previous
import functools

import jax
import jax.numpy as jnp
from jax import lax
from jax.experimental import pallas as pl
from jax.experimental.pallas import tpu as pltpu
from jax.experimental.pallas import tpu_sc as plsc

N = 2048
E = 32768
F_IN = 128
HEADS = 2
NHID = 64
NCLS = 16
NGAT = 13
NPSI = 3
E_TOT = E + N

RB = 256


def _mmu_body(u_ref, r_ref, o_ref):
    o_ref[...] = jnp.dot(u_ref[...], jnp.abs(r_ref[...]),
                         preferred_element_type=jnp.float32)


def _mm_u_abs(U, rhs):
    F = rhs.shape[1]
    nc = F // 128
    grid = (N // RB, nc)
    return pl.pallas_call(
        _mmu_body,
        grid=grid,
        in_specs=[
            pl.BlockSpec((RB, N), lambda r, c: (r, 0)),
            pl.BlockSpec((N, 128), lambda r, c: (0, c)),
        ],
        out_specs=pl.BlockSpec((RB, 128), lambda r, c: (r, c)),
        out_shape=jax.ShapeDtypeStruct((N, F), jnp.float32),
    )(U, rhs)


def _mmp_body(p_ref, r_ref, o_ref):
    o_ref[...] = jnp.dot(p_ref[0], jnp.abs(r_ref[...]),
                         preferred_element_type=jnp.float32)


def _mm_psi_abs(psi, rhs):
    F = rhs.shape[1]
    nc = F // 128
    grid = (NPSI, N // RB, nc)
    return pl.pallas_call(
        _mmp_body,
        grid=grid,
        in_specs=[
            pl.BlockSpec((1, RB, N), lambda j, r, c: (j, r, 0)),
            pl.BlockSpec((N, 128), lambda j, r, c: (0, c)),
        ],
        out_specs=pl.BlockSpec((RB, 128), lambda j, r, c: (r, j * nc + c)),
        out_shape=jax.ShapeDtypeStruct((N, NPSI * F), jnp.float32),
    )(psi, rhs)


def _prep_body(coef_ref, w_ref, asrc_ref, adst_ref, h_ref, a_ref, b_ref, c_ref):
    r = pl.program_id(1)
    h = jnp.dot(coef_ref[...], w_ref[0], preferred_element_type=jnp.float32)
    h_ref[0] = h
    a_s = asrc_ref[0]
    a_d = adst_ref[0]
    h2 = h.reshape(RB, HEADS, F_IN)
    al_s = jnp.sum(h2 * a_s[None], axis=-1)
    al_d = jnp.sum(h2 * a_d[None], axis=-1)
    a_ref[0] = al_s
    b_ref[0] = al_d

    ms = jnp.broadcast_to(jnp.max(al_s, axis=0)[:, None], (HEADS, 16))
    md = jnp.broadcast_to(jnp.max(al_d, axis=0)[:, None], (HEADS, 16))

    @pl.when(r == 0)
    def _():
        c_ref[0, :, 0, :] = ms
        c_ref[0, :, 1, :] = md

    @pl.when(r != 0)
    def _():
        c_ref[0, :, 0, :] = jnp.maximum(c_ref[0, :, 0, :], ms)
        c_ref[0, :, 1, :] = jnp.maximum(c_ref[0, :, 1, :], md)


def _gat_prep(coef_all, gat_W, att_src, att_dst):
    grid = (NGAT, N // RB)
    return pl.pallas_call(
        _prep_body,
        grid=grid,
        in_specs=[
            pl.BlockSpec((RB, F_IN), lambda l, r: (r, l)),
            pl.BlockSpec((1, F_IN, HEADS * F_IN), lambda l, r: (l, 0, 0)),
            pl.BlockSpec((1, HEADS, F_IN), lambda l, r: (l, 0, 0)),
            pl.BlockSpec((1, HEADS, F_IN), lambda l, r: (l, 0, 0)),
        ],
        out_specs=[
            pl.BlockSpec((1, RB, HEADS * F_IN), lambda l, r: (l, r, 0)),
            pl.BlockSpec((1, RB, HEADS), lambda l, r: (l, r, 0)),
            pl.BlockSpec((1, RB, HEADS), lambda l, r: (l, r, 0)),
            pl.BlockSpec((1, HEADS, 2, 16), lambda l, r: (l, 0, 0, 0)),
        ],
        out_shape=[
            jax.ShapeDtypeStruct((NGAT, N, HEADS * F_IN), jnp.float32),
            jax.ShapeDtypeStruct((NGAT, N, HEADS), jnp.float32),
            jax.ShapeDtypeStruct((NGAT, N, HEADS), jnp.float32),
            jax.ShapeDtypeStruct((NGAT, HEADS, 2, 16), jnp.float32),
        ],
    )(coef_all, gat_W, att_src, att_dst)


def _elu(x):
    return jnp.where(x > 0, x, jnp.exp(jnp.minimum(x, 0.0)) - 1.0)


def _head_body(acc_ref, s_ref, bias_ref, mw_ref, mb_ref, ow_ref, ob_ref, o_ref):
    ys = []
    for l in range(NGAT):
        a = acc_ref[l]
        s = s_ref[l]
        d0 = jnp.broadcast_to(s[:, 0:1], (RB, F_IN)) + 1e-16
        d1 = jnp.broadcast_to(s[:, 1:2], (RB, F_IN)) + 1e-16
        g = a / jnp.concatenate([d0, d1], axis=1) + bias_ref[l][None, :]
        y = jnp.dot(_elu(g), mw_ref[l], preferred_element_type=jnp.float32)
        ys.append(y + mb_ref[l][None, :])
    yo = jnp.concatenate(ys, axis=1)
    logits = jnp.dot(_elu(yo), ow_ref[...],
                     preferred_element_type=jnp.float32) + ob_ref[...][None, :]
    m = jnp.max(logits, axis=1, keepdims=True)
    z = logits - m
    o_ref[...] = z - jnp.log(jnp.sum(jnp.exp(z), axis=1, keepdims=True))


def _head(acc_all, s_all, gat_bias, mlp_W, mlp_b, out_W, out_b):
    grid = (N // RB,)
    return pl.pallas_call(
        _head_body,
        grid=grid,
        in_specs=[
            pl.BlockSpec((NGAT, RB, HEADS * F_IN), lambda r: (0, r, 0)),
            pl.BlockSpec((NGAT, RB, 16), lambda r: (0, r, 0)),
            pl.BlockSpec((NGAT, HEADS * F_IN), lambda r: (0, 0)),
            pl.BlockSpec((NGAT, HEADS * F_IN, NHID), lambda r: (0, 0, 0)),
            pl.BlockSpec((NGAT, NHID), lambda r: (0, 0)),
            pl.BlockSpec((NGAT * NHID, NCLS), lambda r: (0, 0)),
            pl.BlockSpec((NCLS,), lambda r: (0,)),
        ],
        out_specs=pl.BlockSpec((RB, NCLS), lambda r: (r, 0)),
        out_shape=jax.ShapeDtypeStruct((N, NCLS), jnp.float32),
    )(acc_all, s_all, gat_bias, mlp_W, mlp_b, out_W, out_b)


NS = 16
NLC = 7
EPT = 2560
K = 64
NCH = EPT // K
RPT = N // NS


def _edge_sc(h2, asrc_all, adst_all, cmax_all, src_p, dstg_p, dstr2d, mask_p,
             z256, z16):
    mesh = plsc.VectorSubcoreMesh(core_axis_name="c", subcore_axis_name="s")
    HF = HEADS * F_IN

    @functools.partial(
        pl.kernel, mesh=mesh,
        compiler_params=pltpu.CompilerParams(needs_layout_passes=False),
        out_type=[jax.ShapeDtypeStruct((NGAT * N, HF), jnp.float32),
                  jax.ShapeDtypeStruct((NGAT * N, 16), jnp.float32)],
        scratch_types=[
            pltpu.VMEM((RPT, HF), jnp.float32),
            pltpu.VMEM((RPT, 16), jnp.float32),
            pltpu.VMEM((EPT,), jnp.int32),
            pltpu.VMEM((EPT,), jnp.int32),
            pltpu.VMEM((EPT,), jnp.int32),
            pltpu.VMEM((EPT,), jnp.float32),
            pltpu.VMEM((EPT,), jnp.int32),
            pltpu.VMEM((N,), jnp.float32),
            pltpu.VMEM((N,), jnp.float32),
            pltpu.VMEM((N,), jnp.float32),
            pltpu.VMEM((N,), jnp.float32),
            pltpu.VMEM((HEADS, 2, 16), jnp.float32),
            pltpu.VMEM((K, HF), jnp.float32),
            pltpu.VMEM((K,), jnp.float32),
            pltpu.VMEM((K,), jnp.float32),
            pltpu.SemaphoreType.DMA,
        ])
    def k(src_hbm, dstg_hbm, dstr_hbm, mask_hbm, h2_hbm, asrc_hbm, adst_hbm,
          cmax_hbm, z256_hbm, z16_hbm, acc_out, s_out, acc_v, s_v, src_loc,
          dst_loc, dstr_loc, mask_loc, src2, asrc0_v, asrc1_v, adst0_v,
          adst1_v, cmax_v, rows, wbuf0, wbuf1, sem):
        sid = lax.axis_index("s")
        cid = lax.axis_index("c")
        zi = jnp.zeros((16,), jnp.int32)
        zf = jnp.zeros((16,), jnp.float32)
        lane = lax.iota(jnp.int32, 16)

        pltpu.sync_copy(src_hbm.at[sid], src_loc)
        pltpu.sync_copy(dstg_hbm.at[sid], dst_loc)
        pltpu.sync_copy(dstr_hbm.at[sid], dstr_loc)
        pltpu.sync_copy(mask_hbm.at[sid], mask_loc)

        for i in range(NLC):
            l = cid * NLC + i

            @pl.when(l < NGAT)
            def _layer():
                off = l * N
                pltpu.sync_copy(z256_hbm, acc_v)
                pltpu.sync_copy(z16_hbm, s_v)
                pltpu.sync_copy(asrc_hbm.at[l, 0], asrc0_v)
                pltpu.sync_copy(asrc_hbm.at[l, 1], asrc1_v)
                pltpu.sync_copy(adst_hbm.at[l, 0], adst0_v)
                pltpu.sync_copy(adst_hbm.at[l, 1], adst1_v)
                pltpu.sync_copy(cmax_hbm.at[l], cmax_v)
                c0 = jnp.maximum(zf, cmax_v[0, 0] + cmax_v[0, 1])
                c1 = jnp.maximum(zf, cmax_v[1, 0] + cmax_v[1, 1])

                def sb(t, carry):
                    src2[pl.ds(t * 16, 16)] = src_loc[pl.ds(t * 16, 16)] + off
                    return carry
                lax.fori_loop(0, EPT // 16, sb, 0)

                def cb(ci, carry):
                    base = ci * K
                    for g in range(K // 16):
                        o = base + g * 16
                        sv = src_loc[pl.ds(o, 16)]
                        dv = dst_loc[pl.ds(o, 16)]
                        m = mask_loc[pl.ds(o, 16)]
                        a0 = plsc.load_gather(asrc0_v, [sv])
                        a1 = plsc.load_gather(asrc1_v, [sv])
                        b0 = plsc.load_gather(adst0_v, [dv])
                        b1 = plsc.load_gather(adst1_v, [dv])
                        z0 = a0 + b0
                        z1 = a1 + b1
                        w0 = m * jnp.exp(jnp.where(z0 > 0, z0, 0.2 * z0) - c0)
                        w1 = m * jnp.exp(jnp.where(z1 > 0, z1, 0.2 * z1) - c1)
                        wbuf0[pl.ds(g * 16, 16)] = w0
                        wbuf1[pl.ds(g * 16, 16)] = w1
                    pltpu.async_copy(
                        h2_hbm.at[src2.at[pl.ds(base, K)]], rows, sem).wait()

                    def eb(e2, carry2):
                        eg = base + e2
                        s0 = plsc.load_gather(wbuf0, [zi + e2])
                        s1 = plsc.load_gather(wbuf1, [zi + e2])
                        dsp = plsc.load_gather(dstr_loc, [zi + eg])
                        for j in range(8):
                            plsc.addupdate_scatter(
                                acc_v, [dsp, lane + j * 16],
                                rows[e2, pl.ds(j * 16, 16)] * s0)
                        for j in range(8, 16):
                            plsc.addupdate_scatter(
                                acc_v, [dsp, lane + j * 16],
                                rows[e2, pl.ds(j * 16, 16)] * s1)
                        plsc.addupdate_scatter(
                            s_v, [dsp, lane],
                            jnp.where(lane == 0, s0,
                                      jnp.where(lane == 1, s1, 0.0)))
                        return carry2
                    lax.fori_loop(0, K, eb, 0)
                    return carry
                lax.fori_loop(0, NCH, cb, 0)

                pltpu.sync_copy(
                    acc_v, acc_out.at[pl.ds(off + sid * RPT, RPT)])
                pltpu.sync_copy(
                    s_v, s_out.at[pl.ds(off + sid * RPT, RPT)])

    return k(src_p, dstg_p, dstr2d, mask_p, h2, asrc_all, adst_all, cmax_all,
             z256, z16)


def _edge_phase(h_all, asrc_all, adst_all, cmax_all, src, dst):
    C = jnp.maximum(0.0, cmax_all[:, :, 0, 0] + cmax_all[:, :, 1, 0])
    z = asrc_all[:, src, :] + adst_all[:, dst, :]
    e = jnp.where(z > 0, z, 0.2 * z)
    w = jnp.exp(e - C[:, None, :])
    s = jax.vmap(lambda wl: jax.ops.segment_sum(wl, dst, num_segments=N))(w)
    s16 = jnp.concatenate(
        [s, jnp.zeros((NGAT, N, 16 - HEADS), jnp.float32)], axis=-1)
    hh = h_all.reshape(NGAT, N, HEADS, F_IN)
    msg = hh[:, src] * w[..., None]
    acc = jax.vmap(lambda ml: jax.ops.segment_sum(ml, dst, num_segments=N))(msg)
    return acc.reshape(NGAT, N, HEADS * F_IN), s16


_PERM9 = [0, 3, 6, 1, 4, 7, 2, 5, 8]


def kernel(x, edge_index, U, psi, gat_W, gat_att_src, gat_att_dst, gat_bias,
           mlp_W, mlp_b, out_W, out_b):
    loop = jnp.arange(N, dtype=edge_index.dtype)
    ei = jnp.concatenate([edge_index, jnp.stack([loop, loop])], axis=1)
    src, dst = ei[0], ei[1]

    low = _mm_u_abs(U, x)
    T = _mm_psi_abs(psi, x)
    coef1 = _mm_u_abs(U, T)
    T2 = _mm_psi_abs(psi, T)
    coef2 = _mm_u_abs(U, T2)
    coef2r = coef2.reshape(N, 9, F_IN)[:, jnp.array(_PERM9), :].reshape(N, 9 * F_IN)
    coef_all = jnp.concatenate([low, coef1, coef2r], axis=1)

    h_all, asrc_all, adst_all, cmax_all = _gat_prep(
        coef_all, gat_W, gat_att_src, gat_att_dst)

    order = jnp.argsort(dst)
    dsts = dst[order]
    srcs = src[order]
    owner = dsts // RPT
    starts = jnp.concatenate(
        [jnp.zeros((1,), jnp.int32),
         jnp.cumsum(jnp.bincount(owner, length=NS)).astype(jnp.int32)[:-1]])
    pos = jnp.arange(E_TOT, dtype=jnp.int32) - starts[owner]
    flat = jnp.where(pos < EPT, owner * EPT + pos, NS * EPT)
    src_p = jnp.zeros((NS * EPT + 1,), jnp.int32).at[flat].set(srcs)[:-1]
    dstg_p = jnp.zeros((NS * EPT + 1,), jnp.int32).at[flat].set(dsts)[:-1]
    mask_p = jnp.zeros((NS * EPT + 1,), jnp.float32).at[flat].set(1.0)[:-1]
    dstr_p = (dstg_p - (jnp.arange(NS * EPT, dtype=jnp.int32) // EPT) * RPT
              ) * mask_p.astype(jnp.int32)

    h2 = h_all.reshape(NGAT * N, HEADS * F_IN)
    z256 = jnp.zeros((RPT, HEADS * F_IN), jnp.float32)
    z16 = jnp.zeros((RPT, 16), jnp.float32)
    acc2, s2 = _edge_sc(h2, asrc_all.transpose(0, 2, 1),
                        adst_all.transpose(0, 2, 1), cmax_all,
                        src_p.reshape(NS, EPT), dstg_p.reshape(NS, EPT),
                        dstr_p.reshape(NS, EPT), mask_p.reshape(NS, EPT),
                        z256, z16)
    acc_all = acc2.reshape(NGAT, N, HEADS * F_IN)
    s_all = s2.reshape(NGAT, N, 16)

    return _head(acc_all, s_all, gat_bias, mlp_W, mlp_b, out_W, out_b)


# speedup vs baseline: 12.4013x; 1.1266x over previous
import functools

import jax
import jax.numpy as jnp
from jax import lax
from jax.experimental import pallas as pl
from jax.experimental.pallas import tpu as pltpu
from jax.experimental.pallas import tpu_sc as plsc

N = 2048
E = 32768
F_IN = 128
HEADS = 2
NHID = 64
NCLS = 16
NGAT = 13
NPSI = 3
E_TOT = E + N

RB = 256


def _mmu_body(u_ref, r_ref, o_ref):
    o_ref[...] = jnp.dot(u_ref[...], jnp.abs(r_ref[...]),
                         preferred_element_type=jnp.float32)


def _mm_u_abs(U, rhs):
    F = rhs.shape[1]
    nc = F // 128
    grid = (N // RB, nc)
    return pl.pallas_call(
        _mmu_body,
        grid=grid,
        in_specs=[
            pl.BlockSpec((RB, N), lambda r, c: (r, 0)),
            pl.BlockSpec((N, 128), lambda r, c: (0, c)),
        ],
        out_specs=pl.BlockSpec((RB, 128), lambda r, c: (r, c)),
        out_shape=jax.ShapeDtypeStruct((N, F), jnp.float32),
    )(U, rhs)


def _mmp_body(p_ref, r_ref, o_ref):
    o_ref[...] = jnp.dot(p_ref[0], jnp.abs(r_ref[...]),
                         preferred_element_type=jnp.float32)


def _mm_psi_abs(psi, rhs):
    F = rhs.shape[1]
    nc = F // 128
    grid = (NPSI, N // RB, nc)
    return pl.pallas_call(
        _mmp_body,
        grid=grid,
        in_specs=[
            pl.BlockSpec((1, RB, N), lambda j, r, c: (j, r, 0)),
            pl.BlockSpec((N, 128), lambda j, r, c: (0, c)),
        ],
        out_specs=pl.BlockSpec((RB, 128), lambda j, r, c: (r, j * nc + c)),
        out_shape=jax.ShapeDtypeStruct((N, NPSI * F), jnp.float32),
    )(psi, rhs)


def _prep_body(coef_ref, w_ref, asrc_ref, adst_ref, h_ref, a_ref, b_ref, c_ref):
    r = pl.program_id(1)
    h = jnp.dot(coef_ref[...], w_ref[0], preferred_element_type=jnp.float32)
    h_ref[0] = h
    a_s = asrc_ref[0]
    a_d = adst_ref[0]
    h2 = h.reshape(RB, HEADS, F_IN)
    al_s = jnp.sum(h2 * a_s[None], axis=-1)
    al_d = jnp.sum(h2 * a_d[None], axis=-1)
    a_ref[0] = al_s
    b_ref[0] = al_d

    ms = jnp.broadcast_to(jnp.max(al_s, axis=0)[:, None], (HEADS, 16))
    md = jnp.broadcast_to(jnp.max(al_d, axis=0)[:, None], (HEADS, 16))

    @pl.when(r == 0)
    def _():
        c_ref[0, :, 0, :] = ms
        c_ref[0, :, 1, :] = md

    @pl.when(r != 0)
    def _():
        c_ref[0, :, 0, :] = jnp.maximum(c_ref[0, :, 0, :], ms)
        c_ref[0, :, 1, :] = jnp.maximum(c_ref[0, :, 1, :], md)


def _gat_prep(coef_all, gat_W, att_src, att_dst):
    grid = (NGAT, N // RB)
    return pl.pallas_call(
        _prep_body,
        grid=grid,
        in_specs=[
            pl.BlockSpec((RB, F_IN), lambda l, r: (r, l)),
            pl.BlockSpec((1, F_IN, HEADS * F_IN), lambda l, r: (l, 0, 0)),
            pl.BlockSpec((1, HEADS, F_IN), lambda l, r: (l, 0, 0)),
            pl.BlockSpec((1, HEADS, F_IN), lambda l, r: (l, 0, 0)),
        ],
        out_specs=[
            pl.BlockSpec((1, RB, HEADS * F_IN), lambda l, r: (l, r, 0)),
            pl.BlockSpec((1, RB, HEADS), lambda l, r: (l, r, 0)),
            pl.BlockSpec((1, RB, HEADS), lambda l, r: (l, r, 0)),
            pl.BlockSpec((1, HEADS, 2, 16), lambda l, r: (l, 0, 0, 0)),
        ],
        out_shape=[
            jax.ShapeDtypeStruct((NGAT, N, HEADS * F_IN), jnp.float32),
            jax.ShapeDtypeStruct((NGAT, N, HEADS), jnp.float32),
            jax.ShapeDtypeStruct((NGAT, N, HEADS), jnp.float32),
            jax.ShapeDtypeStruct((NGAT, HEADS, 2, 16), jnp.float32),
        ],
    )(coef_all, gat_W, att_src, att_dst)


def _elu(x):
    return jnp.where(x > 0, x, jnp.exp(jnp.minimum(x, 0.0)) - 1.0)


def _head_body(acc_ref, s_ref, bias_ref, mw_ref, mb_ref, ow_ref, ob_ref, o_ref):
    ys = []
    for l in range(NGAT):
        a = acc_ref[l]
        s = s_ref[l]
        d0 = jnp.broadcast_to(s[:, 0:1], (RB, F_IN)) + 1e-16
        d1 = jnp.broadcast_to(s[:, 1:2], (RB, F_IN)) + 1e-16
        g = a / jnp.concatenate([d0, d1], axis=1) + bias_ref[l][None, :]
        y = jnp.dot(_elu(g), mw_ref[l], preferred_element_type=jnp.float32)
        ys.append(y + mb_ref[l][None, :])
    yo = jnp.concatenate(ys, axis=1)
    logits = jnp.dot(_elu(yo), ow_ref[...],
                     preferred_element_type=jnp.float32) + ob_ref[...][None, :]
    m = jnp.max(logits, axis=1, keepdims=True)
    z = logits - m
    o_ref[...] = z - jnp.log(jnp.sum(jnp.exp(z), axis=1, keepdims=True))


def _head(acc_all, s_all, gat_bias, mlp_W, mlp_b, out_W, out_b):
    grid = (N // RB,)
    return pl.pallas_call(
        _head_body,
        grid=grid,
        in_specs=[
            pl.BlockSpec((NGAT, RB, HEADS * F_IN), lambda r: (0, r, 0)),
            pl.BlockSpec((NGAT, RB, 16), lambda r: (0, r, 0)),
            pl.BlockSpec((NGAT, HEADS * F_IN), lambda r: (0, 0)),
            pl.BlockSpec((NGAT, HEADS * F_IN, NHID), lambda r: (0, 0, 0)),
            pl.BlockSpec((NGAT, NHID), lambda r: (0, 0)),
            pl.BlockSpec((NGAT * NHID, NCLS), lambda r: (0, 0)),
            pl.BlockSpec((NCLS,), lambda r: (0,)),
        ],
        out_specs=pl.BlockSpec((RB, NCLS), lambda r: (r, 0)),
        out_shape=jax.ShapeDtypeStruct((N, NCLS), jnp.float32),
    )(acc_all, s_all, gat_bias, mlp_W, mlp_b, out_W, out_b)


NS = 16
NLC = 7
EPT = 2560
K = 64
NCH = EPT // K
RPT = N // NS


def _edge_sc(h2, asrc_all, adst_all, cmax_all, src_p, dstg_p, dstr2d, mask_p,
             z256, z16):
    mesh = plsc.VectorSubcoreMesh(core_axis_name="c", subcore_axis_name="s")
    HF = HEADS * F_IN

    @functools.partial(
        pl.kernel, mesh=mesh,
        compiler_params=pltpu.CompilerParams(needs_layout_passes=False),
        out_type=[jax.ShapeDtypeStruct((NGAT * N, HF), jnp.float32),
                  jax.ShapeDtypeStruct((NGAT * N, 16), jnp.float32)],
        scratch_types=[
            pltpu.VMEM((RPT, HF), jnp.float32),
            pltpu.VMEM((RPT, 16), jnp.float32),
            pltpu.VMEM((EPT,), jnp.int32),
            pltpu.VMEM((EPT,), jnp.int32),
            pltpu.VMEM((EPT,), jnp.int32),
            pltpu.VMEM((EPT,), jnp.float32),
            pltpu.VMEM((EPT,), jnp.int32),
            pltpu.VMEM((N,), jnp.float32),
            pltpu.VMEM((N,), jnp.float32),
            pltpu.VMEM((N,), jnp.float32),
            pltpu.VMEM((N,), jnp.float32),
            pltpu.VMEM((HEADS, 2, 16), jnp.float32),
            pltpu.VMEM((K, HF), jnp.float32),
            pltpu.VMEM((K, HF), jnp.float32),
            pltpu.VMEM((K,), jnp.float32),
            pltpu.VMEM((K,), jnp.float32),
            pltpu.SemaphoreType.DMA,
        ])
    def k(src_hbm, dstg_hbm, dstr_hbm, mask_hbm, h2_hbm, asrc_hbm, adst_hbm,
          cmax_hbm, z256_hbm, z16_hbm, acc_out, s_out, acc_v, s_v, src_loc,
          dst_loc, dstr_loc, mask_loc, src2, asrc0_v, asrc1_v, adst0_v,
          adst1_v, cmax_v, rows0, rows1, wbuf0, wbuf1, sem):
        sid = lax.axis_index("s")
        cid = lax.axis_index("c")
        zi = jnp.zeros((16,), jnp.int32)
        zf = jnp.zeros((16,), jnp.float32)
        lane = lax.iota(jnp.int32, 16)

        pltpu.sync_copy(src_hbm.at[sid], src_loc)
        pltpu.sync_copy(dstg_hbm.at[sid], dst_loc)
        pltpu.sync_copy(dstr_hbm.at[sid], dstr_loc)
        pltpu.sync_copy(mask_hbm.at[sid], mask_loc)

        def layer_body(i, carry0):
            l = cid * NLC + i

            @pl.when(l < NGAT)
            def _layer():
                off = l * N
                pltpu.sync_copy(z256_hbm, acc_v)
                pltpu.sync_copy(z16_hbm, s_v)
                pltpu.sync_copy(asrc_hbm.at[l, 0], asrc0_v)
                pltpu.sync_copy(asrc_hbm.at[l, 1], asrc1_v)
                pltpu.sync_copy(adst_hbm.at[l, 0], adst0_v)
                pltpu.sync_copy(adst_hbm.at[l, 1], adst1_v)
                pltpu.sync_copy(cmax_hbm.at[l], cmax_v)
                c0 = jnp.maximum(zf, cmax_v[0, 0] + cmax_v[0, 1])
                c1 = jnp.maximum(zf, cmax_v[1, 0] + cmax_v[1, 1])

                def sb(t, carry):
                    src2[pl.ds(t * 16, 16)] = src_loc[pl.ds(t * 16, 16)] + off
                    return carry
                lax.fori_loop(0, EPT // 16, sb, 0)

                pltpu.async_copy(h2_hbm.at[src2.at[pl.ds(0, K)]], rows0, sem)

                def cb(ci2, carry):
                    for b, (rcur, rnxt) in enumerate(
                            ((rows0, rows1), (rows1, rows0))):
                        ci = ci2 * 2 + b
                        base = ci * K

                        @pl.when(ci + 1 < NCH)
                        def _():
                            pltpu.async_copy(
                                h2_hbm.at[src2.at[pl.ds(base + K, K)]],
                                rnxt, sem)

                        for g in range(K // 16):
                            o = base + g * 16
                            sv = src_loc[pl.ds(o, 16)]
                            dv = dst_loc[pl.ds(o, 16)]
                            m = mask_loc[pl.ds(o, 16)]
                            a0 = plsc.load_gather(asrc0_v, [sv])
                            a1 = plsc.load_gather(asrc1_v, [sv])
                            b0 = plsc.load_gather(adst0_v, [dv])
                            b1 = plsc.load_gather(adst1_v, [dv])
                            z0 = a0 + b0
                            z1 = a1 + b1
                            w0 = m * jnp.exp(
                                jnp.where(z0 > 0, z0, 0.2 * z0) - c0)
                            w1 = m * jnp.exp(
                                jnp.where(z1 > 0, z1, 0.2 * z1) - c1)
                            wbuf0[pl.ds(g * 16, 16)] = w0
                            wbuf1[pl.ds(g * 16, 16)] = w1
                        pltpu.make_async_copy(
                            h2_hbm.at[pl.ds(0, K)], rcur, sem).wait()

                        def eb(e4, carry2):
                            for u in range(2):
                                e2 = e4 * 2 + u
                                s0 = plsc.load_gather(wbuf0, [zi + e2])
                                s1 = plsc.load_gather(wbuf1, [zi + e2])
                                dsp = plsc.load_gather(
                                    dstr_loc, [zi + (base + e2)])
                                for j in range(8):
                                    plsc.addupdate_scatter(
                                        acc_v, [dsp, lane + j * 16],
                                        rcur[e2, pl.ds(j * 16, 16)] * s0)
                                for j in range(8, 16):
                                    plsc.addupdate_scatter(
                                        acc_v, [dsp, lane + j * 16],
                                        rcur[e2, pl.ds(j * 16, 16)] * s1)
                                plsc.addupdate_scatter(
                                    s_v, [dsp, lane],
                                    jnp.where(lane == 0, s0,
                                              jnp.where(lane == 1, s1, 0.0)))
                            return carry2
                        lax.fori_loop(0, K // 2, eb, 0)
                    return carry
                lax.fori_loop(0, NCH // 2, cb, 0)

                pltpu.sync_copy(
                    acc_v, acc_out.at[pl.ds(off + sid * RPT, RPT)])
                pltpu.sync_copy(
                    s_v, s_out.at[pl.ds(off + sid * RPT, RPT)])
            return carry0
        lax.fori_loop(0, NLC, layer_body, 0)

    return k(src_p, dstg_p, dstr2d, mask_p, h2, asrc_all, adst_all, cmax_all,
             z256, z16)


def _edge_phase(h_all, asrc_all, adst_all, cmax_all, src, dst):
    C = jnp.maximum(0.0, cmax_all[:, :, 0, 0] + cmax_all[:, :, 1, 0])
    z = asrc_all[:, src, :] + adst_all[:, dst, :]
    e = jnp.where(z > 0, z, 0.2 * z)
    w = jnp.exp(e - C[:, None, :])
    s = jax.vmap(lambda wl: jax.ops.segment_sum(wl, dst, num_segments=N))(w)
    s16 = jnp.concatenate(
        [s, jnp.zeros((NGAT, N, 16 - HEADS), jnp.float32)], axis=-1)
    hh = h_all.reshape(NGAT, N, HEADS, F_IN)
    msg = hh[:, src] * w[..., None]
    acc = jax.vmap(lambda ml: jax.ops.segment_sum(ml, dst, num_segments=N))(msg)
    return acc.reshape(NGAT, N, HEADS * F_IN), s16


_PERM9 = [0, 3, 6, 1, 4, 7, 2, 5, 8]


def kernel(x, edge_index, U, psi, gat_W, gat_att_src, gat_att_dst, gat_bias,
           mlp_W, mlp_b, out_W, out_b):
    loop = jnp.arange(N, dtype=edge_index.dtype)
    ei = jnp.concatenate([edge_index, jnp.stack([loop, loop])], axis=1)
    src, dst = ei[0], ei[1]

    low = _mm_u_abs(U, x)
    T = _mm_psi_abs(psi, x)
    coef1 = _mm_u_abs(U, T)
    T2 = _mm_psi_abs(psi, T)
    coef2 = _mm_u_abs(U, T2)
    coef2r = coef2.reshape(N, 9, F_IN)[:, jnp.array(_PERM9), :].reshape(N, 9 * F_IN)
    coef_all = jnp.concatenate([low, coef1, coef2r], axis=1)

    h_all, asrc_all, adst_all, cmax_all = _gat_prep(
        coef_all, gat_W, gat_att_src, gat_att_dst)

    order = jnp.argsort(dst)
    dsts = dst[order]
    srcs = src[order]
    owner = dsts // RPT
    starts = jnp.concatenate(
        [jnp.zeros((1,), jnp.int32),
         jnp.cumsum(jnp.bincount(owner, length=NS)).astype(jnp.int32)[:-1]])
    pos = jnp.arange(E_TOT, dtype=jnp.int32) - starts[owner]
    flat = jnp.where(pos < EPT, owner * EPT + pos, NS * EPT)
    src_p = jnp.zeros((NS * EPT + 1,), jnp.int32).at[flat].set(srcs)[:-1]
    dstg_p = jnp.zeros((NS * EPT + 1,), jnp.int32).at[flat].set(dsts)[:-1]
    mask_p = jnp.zeros((NS * EPT + 1,), jnp.float32).at[flat].set(1.0)[:-1]
    dstr_p = (dstg_p - (jnp.arange(NS * EPT, dtype=jnp.int32) // EPT) * RPT
              ) * mask_p.astype(jnp.int32)

    h2 = h_all.reshape(NGAT * N, HEADS * F_IN)
    z256 = jnp.zeros((RPT, HEADS * F_IN), jnp.float32)
    z16 = jnp.zeros((RPT, 16), jnp.float32)
    acc2, s2 = _edge_sc(h2, asrc_all.transpose(0, 2, 1),
                        adst_all.transpose(0, 2, 1), cmax_all,
                        src_p.reshape(NS, EPT), dstg_p.reshape(NS, EPT),
                        dstr_p.reshape(NS, EPT), mask_p.reshape(NS, EPT),
                        z256, z16)
    acc_all = acc2.reshape(NGAT, N, HEADS * F_IN)
    s_all = s2.reshape(NGAT, N, 16)

    return _head(acc_all, s_all, gat_bias, mlp_W, mlp_b, out_W, out_b)


# speedup vs baseline: 12.5477x; 1.0118x over previous
import functools

import jax
import jax.numpy as jnp
from jax import lax
from jax.experimental import pallas as pl
from jax.experimental.pallas import tpu as pltpu
from jax.experimental.pallas import tpu_sc as plsc

N = 2048
E = 32768
F_IN = 128
HEADS = 2
NHID = 64
NCLS = 16
NGAT = 13
NPSI = 3
E_TOT = E + N

RB = 256


def _mmu_body(u_ref, r_ref, o_ref):
    o_ref[...] = jnp.dot(u_ref[...], jnp.abs(r_ref[...]),
                         preferred_element_type=jnp.float32)


def _mm_u_abs(U, rhs):
    F = rhs.shape[1]
    nc = F // 128
    grid = (N // RB, nc)
    return pl.pallas_call(
        _mmu_body,
        grid=grid,
        in_specs=[
            pl.BlockSpec((RB, N), lambda r, c: (r, 0)),
            pl.BlockSpec((N, 128), lambda r, c: (0, c)),
        ],
        out_specs=pl.BlockSpec((RB, 128), lambda r, c: (r, c)),
        out_shape=jax.ShapeDtypeStruct((N, F), jnp.float32),
    )(U, rhs)


def _mmp_body(p_ref, r_ref, o_ref):
    o_ref[...] = jnp.dot(p_ref[0], jnp.abs(r_ref[...]),
                         preferred_element_type=jnp.float32)


def _mm_psi_abs(psi, rhs):
    F = rhs.shape[1]
    nc = F // 128
    grid = (NPSI, N // RB, nc)
    return pl.pallas_call(
        _mmp_body,
        grid=grid,
        in_specs=[
            pl.BlockSpec((1, RB, N), lambda j, r, c: (j, r, 0)),
            pl.BlockSpec((N, 128), lambda j, r, c: (0, c)),
        ],
        out_specs=pl.BlockSpec((RB, 128), lambda j, r, c: (r, j * nc + c)),
        out_shape=jax.ShapeDtypeStruct((N, NPSI * F), jnp.float32),
    )(psi, rhs)


def _prep_body(coef_ref, w_ref, asrc_ref, adst_ref, h_ref, a_ref, b_ref, c_ref):
    r = pl.program_id(1)
    h = jnp.dot(coef_ref[...], w_ref[0], preferred_element_type=jnp.float32)
    h_ref[0] = h
    a_s = asrc_ref[0]
    a_d = adst_ref[0]
    h2 = h.reshape(RB, HEADS, F_IN)
    al_s = jnp.sum(h2 * a_s[None], axis=-1)
    al_d = jnp.sum(h2 * a_d[None], axis=-1)
    a_ref[0] = al_s
    b_ref[0] = al_d

    ms = jnp.broadcast_to(jnp.max(al_s, axis=0)[:, None], (HEADS, 16))
    md = jnp.broadcast_to(jnp.max(al_d, axis=0)[:, None], (HEADS, 16))

    @pl.when(r == 0)
    def _():
        c_ref[0, :, 0, :] = ms
        c_ref[0, :, 1, :] = md

    @pl.when(r != 0)
    def _():
        c_ref[0, :, 0, :] = jnp.maximum(c_ref[0, :, 0, :], ms)
        c_ref[0, :, 1, :] = jnp.maximum(c_ref[0, :, 1, :], md)


def _gat_prep(coef_all, gat_W, att_src, att_dst):
    grid = (NGAT, N // RB)
    return pl.pallas_call(
        _prep_body,
        grid=grid,
        in_specs=[
            pl.BlockSpec((RB, F_IN), lambda l, r: (r, l)),
            pl.BlockSpec((1, F_IN, HEADS * F_IN), lambda l, r: (l, 0, 0)),
            pl.BlockSpec((1, HEADS, F_IN), lambda l, r: (l, 0, 0)),
            pl.BlockSpec((1, HEADS, F_IN), lambda l, r: (l, 0, 0)),
        ],
        out_specs=[
            pl.BlockSpec((1, RB, HEADS * F_IN), lambda l, r: (l, r, 0)),
            pl.BlockSpec((1, RB, HEADS), lambda l, r: (l, r, 0)),
            pl.BlockSpec((1, RB, HEADS), lambda l, r: (l, r, 0)),
            pl.BlockSpec((1, HEADS, 2, 16), lambda l, r: (l, 0, 0, 0)),
        ],
        out_shape=[
            jax.ShapeDtypeStruct((NGAT, N, HEADS * F_IN), jnp.float32),
            jax.ShapeDtypeStruct((NGAT, N, HEADS), jnp.float32),
            jax.ShapeDtypeStruct((NGAT, N, HEADS), jnp.float32),
            jax.ShapeDtypeStruct((NGAT, HEADS, 2, 16), jnp.float32),
        ],
    )(coef_all, gat_W, att_src, att_dst)


def _elu(x):
    return jnp.where(x > 0, x, jnp.exp(jnp.minimum(x, 0.0)) - 1.0)


def _head_body(acc_ref, s_ref, bias_ref, mw_ref, mb_ref, ow_ref, ob_ref, o_ref):
    ys = []
    for l in range(NGAT):
        a = acc_ref[l]
        s = s_ref[l]
        d0 = jnp.broadcast_to(s[:, 0:1], (RB, F_IN)) + 1e-16
        d1 = jnp.broadcast_to(s[:, 1:2], (RB, F_IN)) + 1e-16
        g = a / jnp.concatenate([d0, d1], axis=1) + bias_ref[l][None, :]
        y = jnp.dot(_elu(g), mw_ref[l], preferred_element_type=jnp.float32)
        ys.append(y + mb_ref[l][None, :])
    yo = jnp.concatenate(ys, axis=1)
    logits = jnp.dot(_elu(yo), ow_ref[...],
                     preferred_element_type=jnp.float32) + ob_ref[...][None, :]
    m = jnp.max(logits, axis=1, keepdims=True)
    z = logits - m
    o_ref[...] = z - jnp.log(jnp.sum(jnp.exp(z), axis=1, keepdims=True))


def _head(acc_all, s_all, gat_bias, mlp_W, mlp_b, out_W, out_b):
    grid = (N // RB,)
    return pl.pallas_call(
        _head_body,
        grid=grid,
        in_specs=[
            pl.BlockSpec((NGAT, RB, HEADS * F_IN), lambda r: (0, r, 0)),
            pl.BlockSpec((NGAT, RB, 16), lambda r: (0, r, 0)),
            pl.BlockSpec((NGAT, HEADS * F_IN), lambda r: (0, 0)),
            pl.BlockSpec((NGAT, HEADS * F_IN, NHID), lambda r: (0, 0, 0)),
            pl.BlockSpec((NGAT, NHID), lambda r: (0, 0)),
            pl.BlockSpec((NGAT * NHID, NCLS), lambda r: (0, 0)),
            pl.BlockSpec((NCLS,), lambda r: (0,)),
        ],
        out_specs=pl.BlockSpec((RB, NCLS), lambda r: (r, 0)),
        out_shape=jax.ShapeDtypeStruct((N, NCLS), jnp.float32),
    )(acc_all, s_all, gat_bias, mlp_W, mlp_b, out_W, out_b)


NS = 16
NLC = 7
EPT = 2560
K = 64
NCH = EPT // K
RPT = N // NS


def _edge_sc(h2, asrc_all, adst_all, cmax_all, src_p, dstg_p, dstr2d, mask_p,
             z256, z16):
    mesh = plsc.VectorSubcoreMesh(core_axis_name="c", subcore_axis_name="s")
    HF = HEADS * F_IN

    @functools.partial(
        pl.kernel, mesh=mesh,
        compiler_params=pltpu.CompilerParams(needs_layout_passes=False),
        out_type=[jax.ShapeDtypeStruct((NGAT * N, HF), jnp.float32),
                  jax.ShapeDtypeStruct((NGAT * N, 16), jnp.float32)],
        scratch_types=[
            pltpu.VMEM((RPT, HF), jnp.float32),
            pltpu.VMEM((RPT, 16), jnp.float32),
            pltpu.VMEM((EPT,), jnp.int32),
            pltpu.VMEM((EPT,), jnp.int32),
            pltpu.VMEM((EPT,), jnp.int32),
            pltpu.VMEM((EPT,), jnp.float32),
            pltpu.VMEM((EPT,), jnp.int32),
            pltpu.VMEM((N,), jnp.float32),
            pltpu.VMEM((N,), jnp.float32),
            pltpu.VMEM((N,), jnp.float32),
            pltpu.VMEM((N,), jnp.float32),
            pltpu.VMEM((HEADS, 2, 16), jnp.float32),
            pltpu.VMEM((K, HF), jnp.float32),
            pltpu.VMEM((K, HF), jnp.float32),
            pltpu.VMEM((K,), jnp.float32),
            pltpu.VMEM((K,), jnp.float32),
            pltpu.SemaphoreType.DMA,
        ])
    def k(src_hbm, dstg_hbm, dstr_hbm, mask_hbm, h2_hbm, asrc_hbm, adst_hbm,
          cmax_hbm, z256_hbm, z16_hbm, acc_out, s_out, acc_v, s_v, src_loc,
          dst_loc, dstr_loc, mask_loc, src2, asrc0_v, asrc1_v, adst0_v,
          adst1_v, cmax_v, rows0, rows1, wbuf0, wbuf1, sem):
        sid = lax.axis_index("s")
        cid = lax.axis_index("c")
        zi = jnp.zeros((16,), jnp.int32)
        zf = jnp.zeros((16,), jnp.float32)
        lane = lax.iota(jnp.int32, 16)

        pltpu.sync_copy(src_hbm.at[sid], src_loc)
        pltpu.sync_copy(dstg_hbm.at[sid], dst_loc)
        pltpu.sync_copy(dstr_hbm.at[sid], dstr_loc)
        pltpu.sync_copy(mask_hbm.at[sid], mask_loc)

        def layer_body(i, carry0):
            l = cid * NLC + i

            @pl.when(l < NGAT)
            def _layer():
                off = l * N
                pltpu.sync_copy(z256_hbm, acc_v)
                pltpu.sync_copy(z16_hbm, s_v)
                pltpu.sync_copy(asrc_hbm.at[l, 0], asrc0_v)
                pltpu.sync_copy(asrc_hbm.at[l, 1], asrc1_v)
                pltpu.sync_copy(adst_hbm.at[l, 0], adst0_v)
                pltpu.sync_copy(adst_hbm.at[l, 1], adst1_v)
                pltpu.sync_copy(cmax_hbm.at[l], cmax_v)
                c0 = jnp.maximum(zf, cmax_v[0, 0] + cmax_v[0, 1])
                c1 = jnp.maximum(zf, cmax_v[1, 0] + cmax_v[1, 1])

                def sb(t, carry):
                    src2[pl.ds(t * 16, 16)] = src_loc[pl.ds(t * 16, 16)] + off
                    return carry
                lax.fori_loop(0, EPT // 16, sb, 0)

                pltpu.async_copy(h2_hbm.at[src2.at[pl.ds(0, K)]], rows0, sem)

                def cb(ci2, carry):
                    for b, (rcur, rnxt) in enumerate(
                            ((rows0, rows1), (rows1, rows0))):
                        ci = ci2 * 2 + b
                        base = ci * K

                        @pl.when(ci + 1 < NCH)
                        def _():
                            pltpu.async_copy(
                                h2_hbm.at[src2.at[pl.ds(base + K, K)]],
                                rnxt, sem)

                        for g in range(K // 16):
                            o = base + g * 16
                            sv = src_loc[pl.ds(o, 16)]
                            dv = dst_loc[pl.ds(o, 16)]
                            m = mask_loc[pl.ds(o, 16)]
                            a0 = plsc.load_gather(asrc0_v, [sv])
                            a1 = plsc.load_gather(asrc1_v, [sv])
                            b0 = plsc.load_gather(adst0_v, [dv])
                            b1 = plsc.load_gather(adst1_v, [dv])
                            z0 = a0 + b0
                            z1 = a1 + b1
                            w0 = m * jnp.exp(
                                jnp.where(z0 > 0, z0, 0.2 * z0) - c0)
                            w1 = m * jnp.exp(
                                jnp.where(z1 > 0, z1, 0.2 * z1) - c1)
                            wbuf0[pl.ds(g * 16, 16)] = w0
                            wbuf1[pl.ds(g * 16, 16)] = w1
                        pltpu.make_async_copy(
                            h2_hbm.at[pl.ds(0, K)], rcur, sem).wait()

                        def eb(g2, carry2):
                            gv0 = wbuf0[pl.ds(g2 * 16, 16)]
                            gv1 = wbuf1[pl.ds(g2 * 16, 16)]
                            gd = dstr_loc[pl.ds(base + g2 * 16, 16)]
                            for u in range(16):
                                e2 = g2 * 16 + u
                                ui = zi + u
                                s0 = jnp.take(gv0, ui)
                                s1 = jnp.take(gv1, ui)
                                dsp = jnp.take(gd, ui)
                                for j in range(8):
                                    plsc.addupdate_scatter(
                                        acc_v, [dsp, lane + j * 16],
                                        rcur[e2, pl.ds(j * 16, 16)] * s0)
                                for j in range(8, 16):
                                    plsc.addupdate_scatter(
                                        acc_v, [dsp, lane + j * 16],
                                        rcur[e2, pl.ds(j * 16, 16)] * s1)
                                plsc.addupdate_scatter(
                                    s_v, [dsp, lane],
                                    jnp.where(lane == 0, s0,
                                              jnp.where(lane == 1, s1, 0.0)))
                            return carry2
                        lax.fori_loop(0, K // 16, eb, 0)
                    return carry
                lax.fori_loop(0, NCH // 2, cb, 0)

                pltpu.sync_copy(
                    acc_v, acc_out.at[pl.ds(off + sid * RPT, RPT)])
                pltpu.sync_copy(
                    s_v, s_out.at[pl.ds(off + sid * RPT, RPT)])
            return carry0
        lax.fori_loop(0, NLC, layer_body, 0)

    return k(src_p, dstg_p, dstr2d, mask_p, h2, asrc_all, adst_all, cmax_all,
             z256, z16)


def _edge_phase(h_all, asrc_all, adst_all, cmax_all, src, dst):
    C = jnp.maximum(0.0, cmax_all[:, :, 0, 0] + cmax_all[:, :, 1, 0])
    z = asrc_all[:, src, :] + adst_all[:, dst, :]
    e = jnp.where(z > 0, z, 0.2 * z)
    w = jnp.exp(e - C[:, None, :])
    s = jax.vmap(lambda wl: jax.ops.segment_sum(wl, dst, num_segments=N))(w)
    s16 = jnp.concatenate(
        [s, jnp.zeros((NGAT, N, 16 - HEADS), jnp.float32)], axis=-1)
    hh = h_all.reshape(NGAT, N, HEADS, F_IN)
    msg = hh[:, src] * w[..., None]
    acc = jax.vmap(lambda ml: jax.ops.segment_sum(ml, dst, num_segments=N))(msg)
    return acc.reshape(NGAT, N, HEADS * F_IN), s16


_PERM9 = [0, 3, 6, 1, 4, 7, 2, 5, 8]


def kernel(x, edge_index, U, psi, gat_W, gat_att_src, gat_att_dst, gat_bias,
           mlp_W, mlp_b, out_W, out_b):
    loop = jnp.arange(N, dtype=edge_index.dtype)
    ei = jnp.concatenate([edge_index, jnp.stack([loop, loop])], axis=1)
    src, dst = ei[0], ei[1]

    low = _mm_u_abs(U, x)
    T = _mm_psi_abs(psi, x)
    coef1 = _mm_u_abs(U, T)
    T2 = _mm_psi_abs(psi, T)
    coef2 = _mm_u_abs(U, T2)
    coef2r = coef2.reshape(N, 9, F_IN)[:, jnp.array(_PERM9), :].reshape(N, 9 * F_IN)
    coef_all = jnp.concatenate([low, coef1, coef2r], axis=1)

    h_all, asrc_all, adst_all, cmax_all = _gat_prep(
        coef_all, gat_W, gat_att_src, gat_att_dst)

    order = jnp.argsort(dst)
    dsts = dst[order]
    srcs = src[order]
    owner = dsts // RPT
    starts = jnp.concatenate(
        [jnp.zeros((1,), jnp.int32),
         jnp.cumsum(jnp.bincount(owner, length=NS)).astype(jnp.int32)[:-1]])
    pos = jnp.arange(E_TOT, dtype=jnp.int32) - starts[owner]
    flat = jnp.where(pos < EPT, owner * EPT + pos, NS * EPT)
    src_p = jnp.zeros((NS * EPT + 1,), jnp.int32).at[flat].set(srcs)[:-1]
    dstg_p = jnp.zeros((NS * EPT + 1,), jnp.int32).at[flat].set(dsts)[:-1]
    mask_p = jnp.zeros((NS * EPT + 1,), jnp.float32).at[flat].set(1.0)[:-1]
    dstr_p = (dstg_p - (jnp.arange(NS * EPT, dtype=jnp.int32) // EPT) * RPT
              ) * mask_p.astype(jnp.int32)

    h2 = h_all.reshape(NGAT * N, HEADS * F_IN)
    z256 = jnp.zeros((RPT, HEADS * F_IN), jnp.float32)
    z16 = jnp.zeros((RPT, 16), jnp.float32)
    acc2, s2 = _edge_sc(h2, asrc_all.transpose(0, 2, 1),
                        adst_all.transpose(0, 2, 1), cmax_all,
                        src_p.reshape(NS, EPT), dstg_p.reshape(NS, EPT),
                        dstr_p.reshape(NS, EPT), mask_p.reshape(NS, EPT),
                        z256, z16)
    acc_all = acc2.reshape(NGAT, N, HEADS * F_IN)
    s_all = s2.reshape(NGAT, N, 16)

    return _head(acc_all, s_all, gat_bias, mlp_W, mlp_b, out_W, out_b)


# speedup vs baseline: 13.4479x; 1.0717x over previous
import functools

import jax
import jax.numpy as jnp
from jax import lax
from jax.experimental import pallas as pl
from jax.experimental.pallas import tpu as pltpu
from jax.experimental.pallas import tpu_sc as plsc

N = 2048
E = 32768
F_IN = 128
HEADS = 2
NHID = 64
NCLS = 16
NGAT = 13
NPSI = 3
E_TOT = E + N

RB = 256


def _mmu_body(u_ref, r_ref, o_ref):
    o_ref[...] = jnp.dot(u_ref[...], jnp.abs(r_ref[...]),
                         preferred_element_type=jnp.float32)


def _mm_u_abs(U, rhs):
    F = rhs.shape[1]
    nc = F // 128
    grid = (N // RB, nc)
    return pl.pallas_call(
        _mmu_body,
        grid=grid,
        in_specs=[
            pl.BlockSpec((RB, N), lambda r, c: (r, 0)),
            pl.BlockSpec((N, 128), lambda r, c: (0, c)),
        ],
        out_specs=pl.BlockSpec((RB, 128), lambda r, c: (r, c)),
        out_shape=jax.ShapeDtypeStruct((N, F), jnp.float32),
    )(U, rhs)


def _mmp_body(p_ref, r_ref, o_ref):
    o_ref[...] = jnp.dot(p_ref[0], jnp.abs(r_ref[...]),
                         preferred_element_type=jnp.float32)


def _mm_psi_abs(psi, rhs):
    F = rhs.shape[1]
    nc = F // 128
    grid = (NPSI, N // RB, nc)
    return pl.pallas_call(
        _mmp_body,
        grid=grid,
        in_specs=[
            pl.BlockSpec((1, RB, N), lambda j, r, c: (j, r, 0)),
            pl.BlockSpec((N, 128), lambda j, r, c: (0, c)),
        ],
        out_specs=pl.BlockSpec((RB, 128), lambda j, r, c: (r, j * nc + c)),
        out_shape=jax.ShapeDtypeStruct((N, NPSI * F), jnp.float32),
    )(psi, rhs)


def _prep_body(coef_ref, w_ref, asrc_ref, adst_ref, h_ref, a_ref, b_ref, c_ref):
    r = pl.program_id(1)
    h = jnp.dot(coef_ref[...], w_ref[0], preferred_element_type=jnp.float32)
    h_ref[0] = h
    a_s = asrc_ref[0]
    a_d = adst_ref[0]
    h2 = h.reshape(RB, HEADS, F_IN)
    al_s = jnp.sum(h2 * a_s[None], axis=-1)
    al_d = jnp.sum(h2 * a_d[None], axis=-1)
    a_ref[0] = al_s
    b_ref[0] = al_d

    ms = jnp.broadcast_to(jnp.max(al_s, axis=0)[:, None], (HEADS, 16))
    md = jnp.broadcast_to(jnp.max(al_d, axis=0)[:, None], (HEADS, 16))

    @pl.when(r == 0)
    def _():
        c_ref[0, :, 0, :] = ms
        c_ref[0, :, 1, :] = md

    @pl.when(r != 0)
    def _():
        c_ref[0, :, 0, :] = jnp.maximum(c_ref[0, :, 0, :], ms)
        c_ref[0, :, 1, :] = jnp.maximum(c_ref[0, :, 1, :], md)


def _gat_prep(coef_all, gat_W, att_src, att_dst):
    grid = (NGAT, N // RB)
    return pl.pallas_call(
        _prep_body,
        grid=grid,
        in_specs=[
            pl.BlockSpec((RB, F_IN), lambda l, r: (r, l)),
            pl.BlockSpec((1, F_IN, HEADS * F_IN), lambda l, r: (l, 0, 0)),
            pl.BlockSpec((1, HEADS, F_IN), lambda l, r: (l, 0, 0)),
            pl.BlockSpec((1, HEADS, F_IN), lambda l, r: (l, 0, 0)),
        ],
        out_specs=[
            pl.BlockSpec((1, RB, HEADS * F_IN), lambda l, r: (l, r, 0)),
            pl.BlockSpec((1, RB, HEADS), lambda l, r: (l, r, 0)),
            pl.BlockSpec((1, RB, HEADS), lambda l, r: (l, r, 0)),
            pl.BlockSpec((1, HEADS, 2, 16), lambda l, r: (l, 0, 0, 0)),
        ],
        out_shape=[
            jax.ShapeDtypeStruct((NGAT, N, HEADS * F_IN), jnp.float32),
            jax.ShapeDtypeStruct((NGAT, N, HEADS), jnp.float32),
            jax.ShapeDtypeStruct((NGAT, N, HEADS), jnp.float32),
            jax.ShapeDtypeStruct((NGAT, HEADS, 2, 16), jnp.float32),
        ],
    )(coef_all, gat_W, att_src, att_dst)


def _elu(x):
    return jnp.where(x > 0, x, jnp.exp(jnp.minimum(x, 0.0)) - 1.0)


def _head_body(acc_ref, s_ref, bias_ref, mw_ref, mb_ref, ow_ref, ob_ref, o_ref):
    ys = []
    for l in range(NGAT):
        a = acc_ref[l]
        s = s_ref[l]
        d0 = jnp.broadcast_to(s[:, 0:1], (RB, F_IN)) + 1e-16
        d1 = jnp.broadcast_to(s[:, 1:2], (RB, F_IN)) + 1e-16
        g = a / jnp.concatenate([d0, d1], axis=1) + bias_ref[l][None, :]
        y = jnp.dot(_elu(g), mw_ref[l], preferred_element_type=jnp.float32)
        ys.append(y + mb_ref[l][None, :])
    yo = jnp.concatenate(ys, axis=1)
    logits = jnp.dot(_elu(yo), ow_ref[...],
                     preferred_element_type=jnp.float32) + ob_ref[...][None, :]
    m = jnp.max(logits, axis=1, keepdims=True)
    z = logits - m
    o_ref[...] = z - jnp.log(jnp.sum(jnp.exp(z), axis=1, keepdims=True))


def _head(acc_all, s_all, gat_bias, mlp_W, mlp_b, out_W, out_b):
    grid = (N // RB,)
    return pl.pallas_call(
        _head_body,
        grid=grid,
        in_specs=[
            pl.BlockSpec((NGAT, RB, HEADS * F_IN), lambda r: (0, r, 0)),
            pl.BlockSpec((NGAT, RB, 16), lambda r: (0, r, 0)),
            pl.BlockSpec((NGAT, HEADS * F_IN), lambda r: (0, 0)),
            pl.BlockSpec((NGAT, HEADS * F_IN, NHID), lambda r: (0, 0, 0)),
            pl.BlockSpec((NGAT, NHID), lambda r: (0, 0)),
            pl.BlockSpec((NGAT * NHID, NCLS), lambda r: (0, 0)),
            pl.BlockSpec((NCLS,), lambda r: (0,)),
        ],
        out_specs=pl.BlockSpec((RB, NCLS), lambda r: (r, 0)),
        out_shape=jax.ShapeDtypeStruct((N, NCLS), jnp.float32),
    )(acc_all, s_all, gat_bias, mlp_W, mlp_b, out_W, out_b)


NS = 16
NLC = 7
EPT = 2560
K = 32
NCH = EPT // K
RPT = N // NS


def _edge_sc(h2, asrc_all, adst_all, cmax_all, src_p, dstr2d, mask_p,
             z256, z16):
    mesh = plsc.VectorSubcoreMesh(core_axis_name="c", subcore_axis_name="s")
    HF = HEADS * F_IN

    @functools.partial(
        pl.kernel, mesh=mesh,
        compiler_params=pltpu.CompilerParams(needs_layout_passes=False),
        out_type=[jax.ShapeDtypeStruct((NGAT * N, HF), jnp.float32),
                  jax.ShapeDtypeStruct((NGAT * N, 16), jnp.float32)],
        scratch_types=[
            pltpu.VMEM((RPT, HF), jnp.float32),
            pltpu.VMEM((RPT, 16), jnp.float32),
            pltpu.VMEM((EPT,), jnp.int32),
            pltpu.VMEM((EPT,), jnp.int32),
            pltpu.VMEM((EPT,), jnp.float32),
            pltpu.VMEM((EPT,), jnp.int32),
            pltpu.VMEM((N,), jnp.float32),
            pltpu.VMEM((N,), jnp.float32),
            pltpu.VMEM((N,), jnp.float32),
            pltpu.VMEM((N,), jnp.float32),
            pltpu.VMEM((HEADS, 2, 16), jnp.float32),
            pltpu.VMEM((K, 128), jnp.int32),
            pltpu.VMEM((K, 128), jnp.int32),
            pltpu.VMEM((K,), jnp.float32),
            pltpu.VMEM((K,), jnp.float32),
            pltpu.SemaphoreType.DMA,
        ])
    def k(src_hbm, dstr_hbm, mask_hbm, h2_hbm, asrc_hbm, adst_hbm,
          cmax_hbm, z256_hbm, z16_hbm, acc_out, s_out, acc_v, s_v,
          src_loc, dstr_loc, mask_loc, src2, asrc0_v, asrc1_v, adst0_v,
          adst1_v, cmax_v, rows0, rows1, wbuf0, wbuf1, sem):
        sid = lax.axis_index("s")
        cid = lax.axis_index("c")
        zi = jnp.zeros((16,), jnp.int32)
        zf = jnp.zeros((16,), jnp.float32)
        lane = lax.iota(jnp.int32, 16)

        pltpu.sync_copy(src_hbm.at[sid], src_loc)
        pltpu.sync_copy(dstr_hbm.at[sid], dstr_loc)
        pltpu.sync_copy(mask_hbm.at[sid], mask_loc)

        def layer_body(i, carry0):
            l = cid * NLC + i

            @pl.when(l < NGAT)
            def _layer():
                off = l * N
                pltpu.sync_copy(z256_hbm, acc_v)
                pltpu.sync_copy(z16_hbm, s_v)
                pltpu.sync_copy(asrc_hbm.at[l, 0], asrc0_v)
                pltpu.sync_copy(asrc_hbm.at[l, 1], asrc1_v)
                pltpu.sync_copy(adst_hbm.at[l, 0], adst0_v)
                pltpu.sync_copy(adst_hbm.at[l, 1], adst1_v)
                pltpu.sync_copy(cmax_hbm.at[l], cmax_v)
                c0 = jnp.maximum(zf, cmax_v[0, 0] + cmax_v[0, 1])
                c1 = jnp.maximum(zf, cmax_v[1, 0] + cmax_v[1, 1])

                def sb(t, carry):
                    src2[pl.ds(t * 16, 16)] = src_loc[pl.ds(t * 16, 16)] + off
                    return carry
                lax.fori_loop(0, EPT // 16, sb, 0)

                pltpu.async_copy(h2_hbm.at[src2.at[pl.ds(0, K)]], rows0, sem)

                def cb(ci2, carry):
                    for b, (rcur, rnxt) in enumerate(
                            ((rows0, rows1), (rows1, rows0))):
                        ci = ci2 * 2 + b
                        base = ci * K

                        @pl.when(ci + 1 < NCH)
                        def _():
                            pltpu.async_copy(
                                h2_hbm.at[src2.at[pl.ds(base + K, K)]],
                                rnxt, sem)

                        for g in range(K // 16):
                            o = base + g * 16
                            sv = src_loc[pl.ds(o, 16)]
                            dv = dstr_loc[pl.ds(o, 16)] + sid * RPT
                            m = mask_loc[pl.ds(o, 16)]
                            a0 = plsc.load_gather(asrc0_v, [sv])
                            a1 = plsc.load_gather(asrc1_v, [sv])
                            b0 = plsc.load_gather(adst0_v, [dv])
                            b1 = plsc.load_gather(adst1_v, [dv])
                            z0 = a0 + b0
                            z1 = a1 + b1
                            w0 = m * jnp.exp(
                                jnp.where(z0 > 0, z0, 0.2 * z0) - c0)
                            w1 = m * jnp.exp(
                                jnp.where(z1 > 0, z1, 0.2 * z1) - c1)
                            wbuf0[pl.ds(g * 16, 16)] = w0
                            wbuf1[pl.ds(g * 16, 16)] = w1
                        pltpu.make_async_copy(
                            h2_hbm.at[pl.ds(0, K)], rcur, sem).wait()

                        def eb(g2, carry2):
                            gv0 = wbuf0[pl.ds(g2 * 16, 16)]
                            gv1 = wbuf1[pl.ds(g2 * 16, 16)]
                            gd = dstr_loc[pl.ds(base + g2 * 16, 16)]
                            for u in range(16):
                                e2 = g2 * 16 + u
                                ui = zi + u
                                s0 = jnp.take(gv0, ui)
                                s1 = jnp.take(gv1, ui)
                                dsp = jnp.take(gd, ui)
                                for jb in range(8):
                                    sw = s0 if jb < 4 else s1
                                    xi = rcur[e2, pl.ds(jb * 16, 16)]
                                    xb = plsc.bitcast(xi, jnp.bfloat16)
                                    va, vb = plsc.unpack(
                                        xb,
                                        format=plsc.PackFormat.INTERLEAVED)
                                    c = jb * 32
                                    plsc.addupdate_scatter(
                                        acc_v, [dsp, lane + c], va * sw)
                                    plsc.addupdate_scatter(
                                        acc_v, [dsp, lane + c + 16],
                                        vb * sw)
                                plsc.addupdate_scatter(
                                    s_v, [dsp, lane],
                                    jnp.where(lane == 0, s0,
                                              jnp.where(lane == 1, s1, 0.0)))
                            return carry2
                        lax.fori_loop(0, K // 16, eb, 0)
                    return carry
                lax.fori_loop(0, NCH // 2, cb, 0)

                pltpu.sync_copy(
                    acc_v, acc_out.at[pl.ds(off + sid * RPT, RPT)])
                pltpu.sync_copy(
                    s_v, s_out.at[pl.ds(off + sid * RPT, RPT)])
            return carry0
        lax.fori_loop(0, NLC, layer_body, 0)

    return k(src_p, dstr2d, mask_p, h2, asrc_all, adst_all, cmax_all,
             z256, z16)


def _edge_phase(h_all, asrc_all, adst_all, cmax_all, src, dst):
    C = jnp.maximum(0.0, cmax_all[:, :, 0, 0] + cmax_all[:, :, 1, 0])
    z = asrc_all[:, src, :] + adst_all[:, dst, :]
    e = jnp.where(z > 0, z, 0.2 * z)
    w = jnp.exp(e - C[:, None, :])
    s = jax.vmap(lambda wl: jax.ops.segment_sum(wl, dst, num_segments=N))(w)
    s16 = jnp.concatenate(
        [s, jnp.zeros((NGAT, N, 16 - HEADS), jnp.float32)], axis=-1)
    hh = h_all.reshape(NGAT, N, HEADS, F_IN)
    msg = hh[:, src] * w[..., None]
    acc = jax.vmap(lambda ml: jax.ops.segment_sum(ml, dst, num_segments=N))(msg)
    return acc.reshape(NGAT, N, HEADS * F_IN), s16


_PERM9 = [0, 3, 6, 1, 4, 7, 2, 5, 8]


def kernel(x, edge_index, U, psi, gat_W, gat_att_src, gat_att_dst, gat_bias,
           mlp_W, mlp_b, out_W, out_b):
    loop = jnp.arange(N, dtype=edge_index.dtype)
    ei = jnp.concatenate([edge_index, jnp.stack([loop, loop])], axis=1)
    src, dst = ei[0], ei[1]

    low = _mm_u_abs(U, x)
    T = _mm_psi_abs(psi, x)
    coef1 = _mm_u_abs(U, T)
    T2 = _mm_psi_abs(psi, T)
    coef2 = _mm_u_abs(U, T2)
    coef2r = coef2.reshape(N, 9, F_IN)[:, jnp.array(_PERM9), :].reshape(N, 9 * F_IN)
    coef_all = jnp.concatenate([low, coef1, coef2r], axis=1)

    h_all, asrc_all, adst_all, cmax_all = _gat_prep(
        coef_all, gat_W, gat_att_src, gat_att_dst)

    order = jnp.argsort(dst)
    dsts = dst[order]
    srcs = src[order]
    owner = dsts // RPT
    starts = jnp.concatenate(
        [jnp.zeros((1,), jnp.int32),
         jnp.cumsum(jnp.bincount(owner, length=NS)).astype(jnp.int32)[:-1]])
    pos = jnp.arange(E_TOT, dtype=jnp.int32) - starts[owner]
    flat = jnp.where(pos < EPT, owner * EPT + pos, NS * EPT)
    src_p = jnp.zeros((NS * EPT + 1,), jnp.int32).at[flat].set(srcs)[:-1]
    dstg_p = jnp.zeros((NS * EPT + 1,), jnp.int32).at[flat].set(dsts)[:-1]
    mask_p = jnp.zeros((NS * EPT + 1,), jnp.float32).at[flat].set(1.0)[:-1]
    dstr_p = (dstg_p - (jnp.arange(NS * EPT, dtype=jnp.int32) // EPT) * RPT
              ) * mask_p.astype(jnp.int32)

    h2 = lax.bitcast_convert_type(
        h_all.reshape(NGAT * N, 8, 2, 16).transpose(0, 1, 3, 2)
        .astype(jnp.bfloat16).reshape(NGAT * N, 128, 2),
        jnp.int32)
    z256 = jnp.zeros((RPT, HEADS * F_IN), jnp.float32)
    z16 = jnp.zeros((RPT, 16), jnp.float32)
    acc2, s2 = _edge_sc(h2, asrc_all.transpose(0, 2, 1),
                        adst_all.transpose(0, 2, 1), cmax_all,
                        src_p.reshape(NS, EPT),
                        dstr_p.reshape(NS, EPT), mask_p.reshape(NS, EPT),
                        z256, z16)
    acc_all = acc2.reshape(NGAT, N, HEADS * F_IN)
    s_all = s2.reshape(NGAT, N, 16)

    return _head(acc_all, s_all, gat_bias, mlp_W, mlp_b, out_W, out_b)


# speedup vs baseline: 13.6851x; 1.0176x over previous
import functools

import jax
import jax.numpy as jnp
from jax import lax
from jax.experimental import pallas as pl
from jax.experimental.pallas import tpu as pltpu
from jax.experimental.pallas import tpu_sc as plsc

N = 2048
E = 32768
F_IN = 128
HEADS = 2
NHID = 64
NCLS = 16
NGAT = 13
NPSI = 3
E_TOT = E + N

RB = 256


def _mmu_body(u_ref, r_ref, o_ref):
    o_ref[...] = jnp.dot(u_ref[...], jnp.abs(r_ref[...]),
                         preferred_element_type=jnp.float32)


def _mm_u_abs(U, rhs):
    F = rhs.shape[1]
    nc = F // 128
    grid = (N // RB, nc)
    return pl.pallas_call(
        _mmu_body,
        grid=grid,
        in_specs=[
            pl.BlockSpec((RB, N), lambda r, c: (r, 0)),
            pl.BlockSpec((N, 128), lambda r, c: (0, c)),
        ],
        out_specs=pl.BlockSpec((RB, 128), lambda r, c: (r, c)),
        out_shape=jax.ShapeDtypeStruct((N, F), jnp.float32),
    )(U, rhs)


def _mmp_body(p_ref, r_ref, o_ref):
    o_ref[...] = jnp.dot(p_ref[0], jnp.abs(r_ref[...]),
                         preferred_element_type=jnp.float32)


def _mm_psi_abs(psi, rhs):
    F = rhs.shape[1]
    nc = F // 128
    grid = (NPSI, N // RB, nc)
    return pl.pallas_call(
        _mmp_body,
        grid=grid,
        in_specs=[
            pl.BlockSpec((1, RB, N), lambda j, r, c: (j, r, 0)),
            pl.BlockSpec((N, 128), lambda j, r, c: (0, c)),
        ],
        out_specs=pl.BlockSpec((RB, 128), lambda j, r, c: (r, j * nc + c)),
        out_shape=jax.ShapeDtypeStruct((N, NPSI * F), jnp.float32),
    )(psi, rhs)


def _prep_body(coef_ref, w_ref, asrc_ref, adst_ref, h_ref, a_ref, b_ref, c_ref):
    r = pl.program_id(1)
    h = jnp.dot(coef_ref[...], w_ref[0], preferred_element_type=jnp.float32)
    h_ref[0] = h
    a_s = asrc_ref[0]
    a_d = adst_ref[0]
    h2 = h.reshape(RB, HEADS, F_IN)
    al_s = jnp.sum(h2 * a_s[None], axis=-1)
    al_d = jnp.sum(h2 * a_d[None], axis=-1)
    a_ref[0] = al_s
    b_ref[0] = al_d

    ms = jnp.broadcast_to(jnp.max(al_s, axis=0)[:, None], (HEADS, 16))
    md = jnp.broadcast_to(jnp.max(al_d, axis=0)[:, None], (HEADS, 16))

    @pl.when(r == 0)
    def _():
        c_ref[0, :, 0, :] = ms
        c_ref[0, :, 1, :] = md

    @pl.when(r != 0)
    def _():
        c_ref[0, :, 0, :] = jnp.maximum(c_ref[0, :, 0, :], ms)
        c_ref[0, :, 1, :] = jnp.maximum(c_ref[0, :, 1, :], md)


def _gat_prep(coef_all, gat_W, att_src, att_dst):
    grid = (NGAT, N // RB)
    return pl.pallas_call(
        _prep_body,
        grid=grid,
        in_specs=[
            pl.BlockSpec((RB, F_IN), lambda l, r: (r, l)),
            pl.BlockSpec((1, F_IN, HEADS * F_IN), lambda l, r: (l, 0, 0)),
            pl.BlockSpec((1, HEADS, F_IN), lambda l, r: (l, 0, 0)),
            pl.BlockSpec((1, HEADS, F_IN), lambda l, r: (l, 0, 0)),
        ],
        out_specs=[
            pl.BlockSpec((1, RB, HEADS * F_IN), lambda l, r: (l, r, 0)),
            pl.BlockSpec((1, RB, HEADS), lambda l, r: (l, r, 0)),
            pl.BlockSpec((1, RB, HEADS), lambda l, r: (l, r, 0)),
            pl.BlockSpec((1, HEADS, 2, 16), lambda l, r: (l, 0, 0, 0)),
        ],
        out_shape=[
            jax.ShapeDtypeStruct((NGAT, N, HEADS * F_IN), jnp.float32),
            jax.ShapeDtypeStruct((NGAT, N, HEADS), jnp.float32),
            jax.ShapeDtypeStruct((NGAT, N, HEADS), jnp.float32),
            jax.ShapeDtypeStruct((NGAT, HEADS, 2, 16), jnp.float32),
        ],
    )(coef_all, gat_W, att_src, att_dst)


def _elu(x):
    return jnp.where(x > 0, x, jnp.exp(jnp.minimum(x, 0.0)) - 1.0)


def _head_body(acc_ref, s_ref, bias_ref, mw_ref, mb_ref, ow_ref, ob_ref, o_ref):
    ys = []
    for l in range(NGAT):
        a = acc_ref[l]
        s = s_ref[l]
        d0 = jnp.broadcast_to(s[:, 0:1], (RB, F_IN)) + 1e-16
        d1 = jnp.broadcast_to(s[:, 1:2], (RB, F_IN)) + 1e-16
        g = a / jnp.concatenate([d0, d1], axis=1) + bias_ref[l][None, :]
        y = jnp.dot(_elu(g), mw_ref[l], preferred_element_type=jnp.float32)
        ys.append(y + mb_ref[l][None, :])
    yo = jnp.concatenate(ys, axis=1)
    logits = jnp.dot(_elu(yo), ow_ref[...],
                     preferred_element_type=jnp.float32) + ob_ref[...][None, :]
    m = jnp.max(logits, axis=1, keepdims=True)
    z = logits - m
    o_ref[...] = z - jnp.log(jnp.sum(jnp.exp(z), axis=1, keepdims=True))


def _head(acc_all, s_all, gat_bias, mlp_W, mlp_b, out_W, out_b):
    grid = (N // RB,)
    return pl.pallas_call(
        _head_body,
        grid=grid,
        in_specs=[
            pl.BlockSpec((NGAT, RB, HEADS * F_IN), lambda r: (0, r, 0)),
            pl.BlockSpec((NGAT, RB, 16), lambda r: (0, r, 0)),
            pl.BlockSpec((NGAT, HEADS * F_IN), lambda r: (0, 0)),
            pl.BlockSpec((NGAT, HEADS * F_IN, NHID), lambda r: (0, 0, 0)),
            pl.BlockSpec((NGAT, NHID), lambda r: (0, 0)),
            pl.BlockSpec((NGAT * NHID, NCLS), lambda r: (0, 0)),
            pl.BlockSpec((NCLS,), lambda r: (0,)),
        ],
        out_specs=pl.BlockSpec((RB, NCLS), lambda r: (r, 0)),
        out_shape=jax.ShapeDtypeStruct((N, NCLS), jnp.float32),
    )(acc_all, s_all, gat_bias, mlp_W, mlp_b, out_W, out_b)


NS = 16
NLC = 7
EPT = 2560
K = 128
NCH = EPT // K
RPT = N // NS


def _edge_sc(h2, asrc_all, adst_all, cmax_all, src_p, dstr2d, mask_p,
             z256, z16):
    mesh = plsc.VectorSubcoreMesh(core_axis_name="c", subcore_axis_name="s")
    HF = HEADS * F_IN

    @functools.partial(
        pl.kernel, mesh=mesh,
        compiler_params=pltpu.CompilerParams(needs_layout_passes=False),
        out_type=[jax.ShapeDtypeStruct((NGAT * N, HF), jnp.float32),
                  jax.ShapeDtypeStruct((NGAT * N, 16), jnp.float32)],
        scratch_types=[
            pltpu.VMEM((RPT, HF), jnp.float32),
            pltpu.VMEM((RPT, 16), jnp.float32),
            pltpu.VMEM((EPT,), jnp.int32),
            pltpu.VMEM((EPT,), jnp.int32),
            pltpu.VMEM((EPT,), jnp.float32),
            pltpu.VMEM((EPT,), jnp.int32),
            pltpu.VMEM((N,), jnp.float32),
            pltpu.VMEM((N,), jnp.float32),
            pltpu.VMEM((N,), jnp.float32),
            pltpu.VMEM((N,), jnp.float32),
            pltpu.VMEM((HEADS, 2, 16), jnp.float32),
            pltpu.VMEM((K, 128), jnp.int32),
            pltpu.VMEM((K, 128), jnp.int32),
            pltpu.VMEM((K,), jnp.float32),
            pltpu.VMEM((K,), jnp.float32),
            pltpu.SemaphoreType.DMA,
        ])
    def k(src_hbm, dstr_hbm, mask_hbm, h2_hbm, asrc_hbm, adst_hbm,
          cmax_hbm, z256_hbm, z16_hbm, acc_out, s_out, acc_v, s_v,
          src_loc, dstr_loc, mask_loc, src2, asrc0_v, asrc1_v, adst0_v,
          adst1_v, cmax_v, rows0, rows1, wbuf0, wbuf1, sem):
        sid = lax.axis_index("s")
        cid = lax.axis_index("c")
        zi = jnp.zeros((16,), jnp.int32)
        zf = jnp.zeros((16,), jnp.float32)
        lane = lax.iota(jnp.int32, 16)

        pltpu.sync_copy(src_hbm.at[sid], src_loc)
        pltpu.sync_copy(dstr_hbm.at[sid], dstr_loc)
        pltpu.sync_copy(mask_hbm.at[sid], mask_loc)

        def layer_body(i, carry0):
            l = cid * NLC + i

            @pl.when(l < NGAT)
            def _layer():
                off = l * N
                pltpu.sync_copy(z256_hbm, acc_v)
                pltpu.sync_copy(z16_hbm, s_v)
                pltpu.sync_copy(asrc_hbm.at[l, 0], asrc0_v)
                pltpu.sync_copy(asrc_hbm.at[l, 1], asrc1_v)
                pltpu.sync_copy(adst_hbm.at[l, 0], adst0_v)
                pltpu.sync_copy(adst_hbm.at[l, 1], adst1_v)
                pltpu.sync_copy(cmax_hbm.at[l], cmax_v)
                c0 = jnp.maximum(zf, cmax_v[0, 0] + cmax_v[0, 1])
                c1 = jnp.maximum(zf, cmax_v[1, 0] + cmax_v[1, 1])

                def sb(t, carry):
                    src2[pl.ds(t * 16, 16)] = src_loc[pl.ds(t * 16, 16)] + off
                    return carry
                lax.fori_loop(0, EPT // 16, sb, 0)

                pltpu.async_copy(h2_hbm.at[src2.at[pl.ds(0, K)]], rows0, sem)

                def cb(ci2, carry):
                    for b, (rcur, rnxt) in enumerate(
                            ((rows0, rows1), (rows1, rows0))):
                        ci = ci2 * 2 + b
                        base = ci * K

                        @pl.when(ci + 1 < NCH)
                        def _():
                            pltpu.async_copy(
                                h2_hbm.at[src2.at[pl.ds(base + K, K)]],
                                rnxt, sem)

                        for g in range(K // 16):
                            o = base + g * 16
                            sv = src_loc[pl.ds(o, 16)]
                            dv = dstr_loc[pl.ds(o, 16)] + sid * RPT
                            m = mask_loc[pl.ds(o, 16)]
                            a0 = plsc.load_gather(asrc0_v, [sv])
                            a1 = plsc.load_gather(asrc1_v, [sv])
                            b0 = plsc.load_gather(adst0_v, [dv])
                            b1 = plsc.load_gather(adst1_v, [dv])
                            z0 = a0 + b0
                            z1 = a1 + b1
                            w0 = m * jnp.exp(
                                jnp.where(z0 > 0, z0, 0.2 * z0) - c0)
                            w1 = m * jnp.exp(
                                jnp.where(z1 > 0, z1, 0.2 * z1) - c1)
                            wbuf0[pl.ds(g * 16, 16)] = w0
                            wbuf1[pl.ds(g * 16, 16)] = w1
                        pltpu.make_async_copy(
                            h2_hbm.at[pl.ds(0, K)], rcur, sem).wait()

                        def eb(g2, carry2):
                            gv0 = wbuf0[pl.ds(g2 * 16, 16)]
                            gv1 = wbuf1[pl.ds(g2 * 16, 16)]
                            gd = dstr_loc[pl.ds(base + g2 * 16, 16)]
                            for u in range(16):
                                e2 = g2 * 16 + u
                                ui = zi + u
                                s0 = jnp.take(gv0, ui)
                                s1 = jnp.take(gv1, ui)
                                dsp = jnp.take(gd, ui)
                                for jb in range(8):
                                    sw = s0 if jb < 4 else s1
                                    xi = rcur[e2, pl.ds(jb * 16, 16)]
                                    xb = plsc.bitcast(xi, jnp.bfloat16)
                                    va, vb = plsc.unpack(
                                        xb,
                                        format=plsc.PackFormat.INTERLEAVED)
                                    c = jb * 32
                                    plsc.addupdate_scatter(
                                        acc_v, [dsp, lane + c], va * sw)
                                    plsc.addupdate_scatter(
                                        acc_v, [dsp, lane + c + 16],
                                        vb * sw)
                                plsc.addupdate_scatter(
                                    s_v, [dsp, lane],
                                    jnp.where(lane == 0, s0,
                                              jnp.where(lane == 1, s1, 0.0)))
                            return carry2
                        lax.fori_loop(0, K // 16, eb, 0)
                    return carry
                lax.fori_loop(0, NCH // 2, cb, 0)

                pltpu.sync_copy(
                    acc_v, acc_out.at[pl.ds(off + sid * RPT, RPT)])
                pltpu.sync_copy(
                    s_v, s_out.at[pl.ds(off + sid * RPT, RPT)])
            return carry0
        lax.fori_loop(0, NLC, layer_body, 0)

    return k(src_p, dstr2d, mask_p, h2, asrc_all, adst_all, cmax_all,
             z256, z16)


def _edge_phase(h_all, asrc_all, adst_all, cmax_all, src, dst):
    C = jnp.maximum(0.0, cmax_all[:, :, 0, 0] + cmax_all[:, :, 1, 0])
    z = asrc_all[:, src, :] + adst_all[:, dst, :]
    e = jnp.where(z > 0, z, 0.2 * z)
    w = jnp.exp(e - C[:, None, :])
    s = jax.vmap(lambda wl: jax.ops.segment_sum(wl, dst, num_segments=N))(w)
    s16 = jnp.concatenate(
        [s, jnp.zeros((NGAT, N, 16 - HEADS), jnp.float32)], axis=-1)
    hh = h_all.reshape(NGAT, N, HEADS, F_IN)
    msg = hh[:, src] * w[..., None]
    acc = jax.vmap(lambda ml: jax.ops.segment_sum(ml, dst, num_segments=N))(msg)
    return acc.reshape(NGAT, N, HEADS * F_IN), s16


_PERM9 = [0, 3, 6, 1, 4, 7, 2, 5, 8]


def kernel(x, edge_index, U, psi, gat_W, gat_att_src, gat_att_dst, gat_bias,
           mlp_W, mlp_b, out_W, out_b):
    loop = jnp.arange(N, dtype=edge_index.dtype)
    ei = jnp.concatenate([edge_index, jnp.stack([loop, loop])], axis=1)
    src, dst = ei[0], ei[1]

    low = _mm_u_abs(U, x)
    T = _mm_psi_abs(psi, x)
    coef1 = _mm_u_abs(U, T)
    T2 = _mm_psi_abs(psi, T)
    coef2 = _mm_u_abs(U, T2)
    coef2r = coef2.reshape(N, 9, F_IN)[:, jnp.array(_PERM9), :].reshape(N, 9 * F_IN)
    coef_all = jnp.concatenate([low, coef1, coef2r], axis=1)

    h_all, asrc_all, adst_all, cmax_all = _gat_prep(
        coef_all, gat_W, gat_att_src, gat_att_dst)

    order = jnp.argsort(dst)
    dsts = dst[order]
    srcs = src[order]
    owner = dsts // RPT
    starts = jnp.concatenate(
        [jnp.zeros((1,), jnp.int32),
         jnp.cumsum(jnp.bincount(owner, length=NS)).astype(jnp.int32)[:-1]])
    pos = jnp.arange(E_TOT, dtype=jnp.int32) - starts[owner]
    flat = jnp.where(pos < EPT, owner * EPT + pos, NS * EPT)
    src_p = jnp.zeros((NS * EPT + 1,), jnp.int32).at[flat].set(srcs)[:-1]
    dstg_p = jnp.zeros((NS * EPT + 1,), jnp.int32).at[flat].set(dsts)[:-1]
    mask_p = jnp.zeros((NS * EPT + 1,), jnp.float32).at[flat].set(1.0)[:-1]
    dstr_p = (dstg_p - (jnp.arange(NS * EPT, dtype=jnp.int32) // EPT) * RPT
              ) * mask_p.astype(jnp.int32)

    h2 = lax.bitcast_convert_type(
        h_all.reshape(NGAT * N, 8, 2, 16).transpose(0, 1, 3, 2)
        .astype(jnp.bfloat16).reshape(NGAT * N, 128, 2),
        jnp.int32)
    z256 = jnp.zeros((RPT, HEADS * F_IN), jnp.float32)
    z16 = jnp.zeros((RPT, 16), jnp.float32)
    acc2, s2 = _edge_sc(h2, asrc_all.transpose(0, 2, 1),
                        adst_all.transpose(0, 2, 1), cmax_all,
                        src_p.reshape(NS, EPT),
                        dstr_p.reshape(NS, EPT), mask_p.reshape(NS, EPT),
                        z256, z16)
    acc_all = acc2.reshape(NGAT, N, HEADS * F_IN)
    s_all = s2.reshape(NGAT, N, 16)

    return _head(acc_all, s_all, gat_bias, mlp_W, mlp_b, out_W, out_b)
